# Initial kernel scaffold; baseline (speedup 1.0000x reference)
import jax
import jax.numpy as jnp
from jax.experimental import pallas as pl


def kernel(hidden_states, expert_routing_table, router_weights, w13, w2):
    raise NotImplementedError("write your pallas kernel here")


# speedup vs baseline: 1.4316x; 1.4316x over previous
import functools

import jax
import jax.numpy as jnp
from jax.experimental import pallas as pl

T = 2048
D = 1024
DFF = 2048
E = 8
K = 2
BF = 512


def _moe_dense_kernel(rt_ref, rw_ref, x_ref, w13g_ref, w13u_ref, w2_ref, out_ref):
    e = pl.program_id(0)
    j = pl.program_id(1)

    x = x_ref[...]
    gate = jax.lax.dot_general(x, w13g_ref[0], (((1,), (1,)), ((), ())),
                               preferred_element_type=jnp.float32)
    up = jax.lax.dot_general(x, w13u_ref[0], (((1,), (1,)), ((), ())),
                             preferred_element_type=jnp.float32)
    h = (gate * jax.lax.logistic(gate)) * up
    y = jax.lax.dot_general(h, w2_ref[0], (((1,), (1,)), ((), ())),
                            preferred_element_type=jnp.float32)

    rt = rt_ref[...]
    rw = rw_ref[...]
    scale = jnp.sum(jnp.where(rt == e, rw, 0.0), axis=1, keepdims=True)

    @pl.when(jnp.logical_and(e == 0, j == 0))
    def _init():
        out_ref[...] = jnp.zeros_like(out_ref)

    out_ref[...] += scale * y


def kernel(hidden_states, expert_routing_table, router_weights, w13, w2):
    rt = expert_routing_table.astype(jnp.int32)
    grid = (E, DFF // BF)
    out = pl.pallas_call(
        _moe_dense_kernel,
        grid=grid,
        in_specs=[
            pl.BlockSpec((T, K), lambda e, j: (0, 0)),
            pl.BlockSpec((T, K), lambda e, j: (0, 0)),
            pl.BlockSpec((T, D), lambda e, j: (0, 0)),
            pl.BlockSpec((1, BF, D), lambda e, j: (e, j, 0)),
            pl.BlockSpec((1, BF, D), lambda e, j: (e, DFF // BF + j, 0)),
            pl.BlockSpec((1, D, BF), lambda e, j: (e, 0, j)),
        ],
        out_specs=pl.BlockSpec((T, D), lambda e, j: (0, 0)),
        out_shape=jax.ShapeDtypeStruct((T, D), jnp.float32),
    )(rt, router_weights, hidden_states, w13, w13, w2)
    return out
